# baseline (speedup 1.0000x reference)
import functools

import jax
import jax.numpy as jnp
from jax import lax
from jax.experimental import pallas as pl
from jax.experimental.pallas import tpu as pltpu
from jax.experimental.pallas import tpu_sc as plsc

NC = 2
NS = 16
NW = NC * NS
LANES = 16


def _sc_pool(ids_t, table, B, H, E):
    SPW = B // NW
    CH = E // LANES

    mesh = plsc.VectorSubcoreMesh(core_axis_name="c", subcore_axis_name="s")

    NBUF = 5

    @functools.partial(
        pl.kernel,
        mesh=mesh,
        out_type=jax.ShapeDtypeStruct((B, E), jnp.float32),
        scratch_types=(
            [pltpu.VMEM((H, SPW), jnp.int32)]
            + [pltpu.VMEM((SPW, E), jnp.float32) for _ in range(NBUF)]
            + [pltpu.VMEM((SPW, E), jnp.float32)]
            + [pltpu.SemaphoreType.DMA for _ in range(NBUF)]
        ),
    )
    def k(table_hbm, ids_hbm, out_hbm, idx_v, *rest):
        bufs = rest[:NBUF]
        pool_v = rest[NBUF]
        sems = rest[NBUF + 1:]
        wid = lax.axis_index("s") * NC + lax.axis_index("c")
        base = wid * SPW
        pltpu.sync_copy(ids_hbm.at[:, pl.ds(base, SPW)], idx_v)

        def start(r, buf, sem):
            pltpu.async_copy(table_hbm.at[idx_v.at[r]], buf, sem)

        def wait(r, buf, sem):
            pltpu.make_async_copy(table_hbm.at[idx_v.at[r]], buf, sem
                                  ).wait()

        for j in range(NBUF):
            start(j, bufs[j], sems[j])

        @pl.loop(0, SPW)
        def _(s):
            for c in range(CH):
                pool_v[s, pl.ds(c * LANES, LANES)] = jnp.zeros(
                    (LANES,), jnp.float32)

        def accum(buf):
            @plsc.parallel_loop(0, SPW, unroll=8)
            def _(s):
                for c in range(CH):
                    plsc.addupdate(pool_v.at[s, pl.ds(c * LANES, LANES)],
                                   buf[s, pl.ds(c * LANES, LANES)])

        @pl.loop(0, H // NBUF - 1)
        def _(t):
            r0 = NBUF * t
            for j in range(NBUF):
                wait(r0 + j, bufs[j], sems[j])
                accum(bufs[j])
                start(r0 + j + NBUF, bufs[j], sems[j])

        for j in range(NBUF):
            r = H - NBUF + j
            wait(r, bufs[j], sems[j])
            accum(bufs[j])

        pltpu.sync_copy(pool_v, out_hbm.at[pl.ds(base, SPW)])

    return k(table, ids_t)


def _mlp(pooled, W1, b1, W2, b2, B, H, E):
    HID = W1.shape[0]
    OUT = W2.shape[0]
    BB = 4096

    def body(x_ref, w1_ref, b1_ref, w2_ref, b2_ref, o_ref):
        w1s = w1_ref[...] * (1.0 / H)
        h = lax.dot_general(x_ref[...], w1s, (((1,), (1,)), ((), ())),
                            preferred_element_type=jnp.float32)
        h = jnp.maximum(h + b1_ref[...], 0.0)
        ot = lax.dot_general(w2_ref[...], h, (((1,), (1,)), ((), ())),
                             preferred_element_type=jnp.float32)
        o_ref[...] = ot + b2_ref[...]

    out_t = pl.pallas_call(
        body,
        grid=(B // BB,),
        in_specs=[
            pl.BlockSpec((BB, E), lambda i: (i, 0)),
            pl.BlockSpec((HID, E), lambda i: (0, 0)),
            pl.BlockSpec((1, HID), lambda i: (0, 0)),
            pl.BlockSpec((OUT, HID), lambda i: (0, 0)),
            pl.BlockSpec((OUT, 1), lambda i: (0, 0)),
        ],
        out_specs=pl.BlockSpec((OUT, BB), lambda i: (0, i)),
        out_shape=jax.ShapeDtypeStruct((OUT, B), jnp.float32),
    )(pooled, W1, b1.reshape(1, HID), W2, b2.reshape(OUT, 1))
    return out_t.T


def kernel(ids, emb_table, W1, b1, W2, b2):
    B, H = ids.shape
    E = emb_table.shape[1]
    pooled = _sc_pool(ids.astype(jnp.int32).T, emb_table, B, H, E)
    return _mlp(pooled, W1, b1, W2, b2, B, H, E)


# speedup vs baseline: 1.4662x; 1.4662x over previous
import functools

import jax
import jax.numpy as jnp
from jax import lax
from jax.experimental import pallas as pl
from jax.experimental.pallas import tpu as pltpu
from jax.experimental.pallas import tpu_sc as plsc

NC = 2
NS = 16
NW = NC * NS
LANES = 16


def _sc_pool(ids2, table, B, H, E, SPG):
    SPW = B // NW
    CPW = SPW // SPG
    CH = E // LANES

    mesh = plsc.VectorSubcoreMesh(core_axis_name="c", subcore_axis_name="s")

    NBUF = 4

    @functools.partial(
        pl.kernel,
        mesh=mesh,
        out_type=jax.ShapeDtypeStruct((B, E), jnp.float32),
        scratch_types=(
            [pltpu.VMEM((CPW, SPG * H), jnp.int32)]
            + [pltpu.VMEM((SPG * H, E), jnp.float32) for _ in range(NBUF)]
            + [pltpu.VMEM((SPW, E), jnp.float32)]
            + [pltpu.SemaphoreType.DMA for _ in range(NBUF)]
        ),
    )
    def k(table_hbm, ids_hbm, out_hbm, idx_v, *rest):
        bufs = rest[:NBUF]
        pool_v = rest[NBUF]
        sems = rest[NBUF + 1:]
        wid = lax.axis_index("s") * NC + lax.axis_index("c")
        base_chunk = wid * CPW
        pltpu.sync_copy(ids_hbm.at[pl.ds(base_chunk, CPW)], idx_v)

        def start(g, buf, sem):
            pltpu.async_copy(table_hbm.at[idx_v.at[g]], buf, sem)

        def wait(g, buf, sem):
            pltpu.make_async_copy(table_hbm.at[idx_v.at[g]], buf, sem
                                  ).wait()

        def accum(buf, g):
            zero = jnp.zeros((LANES,), jnp.float32)

            def body(r, accs):
                return tuple(
                    accs[s * CH + c] + buf[s * H + r, pl.ds(c * LANES, LANES)]
                    for s in range(SPG) for c in range(CH)
                )
            accs = plsc.parallel_loop(
                0, H, unroll=5, carry=(zero,) * (SPG * CH))(body)
            for s in range(SPG):
                for c in range(CH):
                    pool_v[g * SPG + s, pl.ds(c * LANES, LANES)] = (
                        accs[s * CH + c])

        for j in range(NBUF):
            start(j, bufs[j], sems[j])

        @pl.loop(0, CPW // NBUF - 1)
        def _(t):
            g0 = NBUF * t
            for j in range(NBUF):
                wait(g0 + j, bufs[j], sems[j])
                accum(bufs[j], g0 + j)
                start(g0 + j + NBUF, bufs[j], sems[j])

        for j in range(NBUF):
            g = CPW - NBUF + j
            wait(g, bufs[j], sems[j])
            accum(bufs[j], g)

        pltpu.sync_copy(pool_v, out_hbm.at[pl.ds(wid * SPW, SPW)])

    return k(table, ids2)


def _mlp(pooled, W1, b1, W2, b2, B, H, E):
    HID = W1.shape[0]
    OUT = W2.shape[0]
    BB = 4096

    def body(x_ref, w1_ref, b1_ref, w2_ref, b2_ref, o_ref):
        w1s = w1_ref[...] * (1.0 / H)
        h = lax.dot_general(x_ref[...], w1s, (((1,), (1,)), ((), ())),
                            preferred_element_type=jnp.float32)
        h = jnp.maximum(h + b1_ref[...], 0.0)
        ot = lax.dot_general(w2_ref[...], h, (((1,), (1,)), ((), ())),
                             preferred_element_type=jnp.float32)
        o_ref[...] = ot + b2_ref[...]

    out_t = pl.pallas_call(
        body,
        grid=(B // BB,),
        in_specs=[
            pl.BlockSpec((BB, E), lambda i: (i, 0)),
            pl.BlockSpec((HID, E), lambda i: (0, 0)),
            pl.BlockSpec((1, HID), lambda i: (0, 0)),
            pl.BlockSpec((OUT, HID), lambda i: (0, 0)),
            pl.BlockSpec((OUT, 1), lambda i: (0, 0)),
        ],
        out_specs=pl.BlockSpec((OUT, BB), lambda i: (0, i)),
        out_shape=jax.ShapeDtypeStruct((OUT, B), jnp.float32),
    )(pooled, W1, b1.reshape(1, HID), W2, b2.reshape(OUT, 1))
    return out_t.T


def kernel(ids, emb_table, W1, b1, W2, b2):
    B, H = ids.shape
    E = emb_table.shape[1]
    SPG = 2
    ids2 = ids.astype(jnp.int32).reshape(B // SPG, SPG * H)
    pooled = _sc_pool(ids2, emb_table, B, H, E, SPG)
    return _mlp(pooled, W1, b1, W2, b2, B, H, E)


# speedup vs baseline: 1.4683x; 1.0014x over previous
import functools

import jax
import jax.numpy as jnp
from jax import lax
from jax.experimental import pallas as pl
from jax.experimental.pallas import tpu as pltpu
from jax.experimental.pallas import tpu_sc as plsc

NC = 2
NS = 16
NW = NC * NS
LANES = 16


def _sc_pool(ids2, table, B, H, E, SPG):
    SPW = B // NW
    CPW = SPW // SPG
    CH = E // LANES

    mesh = plsc.VectorSubcoreMesh(core_axis_name="c", subcore_axis_name="s")

    NBUF = 4

    @functools.partial(
        pl.kernel,
        mesh=mesh,
        out_type=jax.ShapeDtypeStruct((B, E), jnp.float32),
        scratch_types=(
            [pltpu.VMEM((CPW, SPG * H), jnp.int32)]
            + [pltpu.VMEM((SPG * H, E), jnp.float32) for _ in range(NBUF)]
            + [pltpu.VMEM((SPW, E), jnp.float32)]
            + [pltpu.SemaphoreType.DMA for _ in range(NBUF)]
        ),
    )
    def k(table_hbm, ids_hbm, out_hbm, idx_v, *rest):
        bufs = rest[:NBUF]
        pool_v = rest[NBUF]
        sems = rest[NBUF + 1:]
        wid = lax.axis_index("s") * NC + lax.axis_index("c")
        base_chunk = wid * CPW
        pltpu.sync_copy(ids_hbm.at[pl.ds(base_chunk, CPW)], idx_v)

        def start(g, buf, sem):
            pltpu.async_copy(table_hbm.at[idx_v.at[g]], buf, sem)

        def wait(g, buf, sem):
            pltpu.make_async_copy(table_hbm.at[idx_v.at[g]], buf, sem
                                  ).wait()

        def accum(buf, g):
            for s in range(SPG):
                zero = jnp.zeros((LANES,), jnp.float32)

                def body(r, accs, s=s):
                    return tuple(
                        accs[c] + buf[r, pl.ds(c * LANES, LANES)]
                        for c in range(CH)
                    )
                accs = plsc.parallel_loop(
                    s * H, (s + 1) * H, unroll=10, carry=(zero,) * CH)(body)
                for c in range(CH):
                    pool_v[g * SPG + s, pl.ds(c * LANES, LANES)] = accs[c]

        for j in range(NBUF):
            start(j, bufs[j], sems[j])

        @pl.loop(0, CPW // NBUF - 1)
        def _(t):
            g0 = NBUF * t
            for j in range(NBUF):
                wait(g0 + j, bufs[j], sems[j])
                accum(bufs[j], g0 + j)
                start(g0 + j + NBUF, bufs[j], sems[j])

        for j in range(NBUF):
            g = CPW - NBUF + j
            wait(g, bufs[j], sems[j])
            accum(bufs[j], g)

        pltpu.sync_copy(pool_v, out_hbm.at[pl.ds(wid * SPW, SPW)])

    return k(table, ids2)


def _mlp(pooled, W1, b1, W2, b2, B, H, E):
    HID = W1.shape[0]
    OUT = W2.shape[0]
    BB = 4096

    def body(x_ref, w1_ref, b1_ref, w2_ref, b2_ref, o_ref):
        w1s = w1_ref[...] * (1.0 / H)
        h = lax.dot_general(x_ref[...], w1s, (((1,), (1,)), ((), ())),
                            preferred_element_type=jnp.float32)
        h = jnp.maximum(h + b1_ref[...], 0.0)
        ot = lax.dot_general(w2_ref[...], h, (((1,), (1,)), ((), ())),
                             preferred_element_type=jnp.float32)
        o_ref[...] = ot + b2_ref[...]

    out_t = pl.pallas_call(
        body,
        grid=(B // BB,),
        in_specs=[
            pl.BlockSpec((BB, E), lambda i: (i, 0)),
            pl.BlockSpec((HID, E), lambda i: (0, 0)),
            pl.BlockSpec((1, HID), lambda i: (0, 0)),
            pl.BlockSpec((OUT, HID), lambda i: (0, 0)),
            pl.BlockSpec((OUT, 1), lambda i: (0, 0)),
        ],
        out_specs=pl.BlockSpec((OUT, BB), lambda i: (0, i)),
        out_shape=jax.ShapeDtypeStruct((OUT, B), jnp.float32),
    )(pooled, W1, b1.reshape(1, HID), W2, b2.reshape(OUT, 1))
    return out_t.T


def kernel(ids, emb_table, W1, b1, W2, b2):
    B, H = ids.shape
    E = emb_table.shape[1]
    SPG = 2
    ids2 = ids.astype(jnp.int32).reshape(B // SPG, SPG * H)
    pooled = _sc_pool(ids2, emb_table, B, H, E, SPG)
    return _mlp(pooled, W1, b1, W2, b2, B, H, E)
